# baseline (speedup 1.0000x reference)
import functools

import jax
import jax.numpy as jnp
from jax import lax
from jax.experimental import pallas as pl
from jax.experimental.pallas import tpu as pltpu
from jax.experimental.pallas import tpu_sc as plsc

D_MODEL = 64
D_PAD = 128
N_ROWS = 1000000
SEQ_I = 4096
SEQ_J = 200
B = SEQ_I * SEQ_J
SCALE = 8.0
LANES = 16

NUM_CORES = 2
NUM_SUBCORES = 16
NW = NUM_CORES * NUM_SUBCORES

PREP_BLK = 1024
PREP_GRID = (N_ROWS + PREP_BLK - 1) // PREP_BLK


def _prep_body(t_ref, o_ref):
    rows_i = lax.broadcasted_iota(jnp.int32, (D_MODEL, D_MODEL), 0)
    cols_i = lax.broadcasted_iota(jnp.int32, (D_MODEL, D_MODEL), 1)
    ident8 = jnp.where(rows_i == cols_i, jnp.float32(SCALE), jnp.float32(0.0))
    rows = lax.dot_general(
        t_ref[...], ident8,
        dimension_numbers=(((0,), (0,)), ((), ())),
        precision=lax.Precision.HIGHEST,
        preferred_element_type=jnp.float32,
    )
    o_ref[...] = jnp.concatenate(
        [rows, jnp.zeros((PREP_BLK, D_PAD - D_MODEL), jnp.float32)], axis=1)


_prep = pl.pallas_call(
    _prep_body,
    grid=(PREP_GRID,),
    in_specs=[pl.BlockSpec((D_MODEL, PREP_BLK), lambda r: (0, r))],
    out_specs=pl.BlockSpec((PREP_BLK, D_PAD), lambda r: (r, 0)),
    out_shape=jax.ShapeDtypeStruct((N_ROWS, D_PAD), jnp.float32),
)


CHUNK = 256
SUB = 128
NSUB = CHUNK // SUB
CHUNKS_PER_J = SEQ_I // CHUNK
N_UNITS = SEQ_J * CHUNKS_PER_J
N_CHUNKS = N_UNITS // NW


def _embed_body(x_hbm, table_hbm, out_hbm,
                idx0, idx1, rows0, rows1,
                isem0, isem1, gsem0, gsem1, ssem0, ssem1):
    wid = lax.axis_index("s") * NUM_CORES + lax.axis_index("c")
    u_base = wid * N_CHUNKS
    idx_v = (idx0, idx1)
    rows_v = (rows0, rows1)
    isem = (isem0, isem1)
    gsem = (gsem0, gsem1)
    ssem = (ssem0, ssem1)

    def unit_j_i0(c):
        g = u_base + c
        return g // CHUNKS_PER_J, (g % CHUNKS_PER_J) * CHUNK

    def fire_gathers(p):
        return [
            pltpu.async_copy(
                table_hbm.at[idx_v[p].at[pl.ds(s * SUB, SUB)]],
                rows_v[p].at[pl.ds(s * SUB, SUB)],
                gsem[p],
            )
            for s in range(NSUB)
        ]

    def wait_gathers(p):
        for s in range(NSUB):
            pltpu.make_async_copy(
                table_hbm.at[idx_v[p].at[pl.ds(s * SUB, SUB)]],
                rows_v[p].at[pl.ds(s * SUB, SUB)],
                gsem[p],
            ).wait()

    def fire_idx(c, p):
        j, i0 = unit_j_i0(c)
        pltpu.async_copy(x_hbm.at[j, pl.ds(i0, CHUNK)], idx_v[p], isem[p])

    def wait_idx(p):
        pltpu.make_async_copy(x_hbm.at[0, pl.ds(0, CHUNK)], idx_v[p],
                              isem[p]).wait()

    def fire_scatter(c, p):
        j, i0 = unit_j_i0(c)
        pltpu.async_copy(rows_v[p],
                         out_hbm.at[pl.ds(j * SEQ_I + i0, CHUNK)],
                         ssem[p])

    def wait_scatter(p):
        pltpu.make_async_copy(rows_v[p],
                              out_hbm.at[pl.ds(0, CHUNK)],
                              ssem[p]).wait()

    def step(c, p, wait_prev_scatter=True, prefetch_gather=True,
             prefetch_idx=True):
        q = 1 - p
        wait_gathers(p)
        if prefetch_gather:
            if wait_prev_scatter:
                wait_scatter(q)
            wait_idx(q)
            fire_gathers(q)
            if prefetch_idx:
                fire_idx(c + 2, p)
        fire_scatter(c, p)

    j0, i00 = unit_j_i0(0)
    pltpu.sync_copy(x_hbm.at[j0, pl.ds(i00, CHUNK)], idx_v[0])
    fire_gathers(0)
    fire_idx(1, 1)

    step(0, 0, wait_prev_scatter=False)
    step(1, 1)

    def pair_body(k, carry):
        c = 2 * k
        step(c, 0)
        step(c + 1, 1)
        return carry

    lax.fori_loop(1, N_CHUNKS // 2 - 1, pair_body, 0)

    step(N_CHUNKS - 2, 0, prefetch_idx=False)
    step(N_CHUNKS - 1, 1, prefetch_gather=False)

    wait_scatter(0)
    wait_scatter(1)


_embed = functools.partial(
    pl.kernel,
    out_type=jax.ShapeDtypeStruct((B, D_PAD), jnp.float32),
    mesh=plsc.VectorSubcoreMesh(
        core_axis_name="c",
        subcore_axis_name="s",
        num_cores=NUM_CORES,
        num_subcores=NUM_SUBCORES,
    ),
    scratch_types=[
        pltpu.VMEM((CHUNK,), jnp.int32),
        pltpu.VMEM((CHUNK,), jnp.int32),
        pltpu.VMEM((CHUNK, D_PAD), jnp.float32),
        pltpu.VMEM((CHUNK, D_PAD), jnp.float32),
        pltpu.SemaphoreType.DMA,
        pltpu.SemaphoreType.DMA,
        pltpu.SemaphoreType.DMA,
        pltpu.SemaphoreType.DMA,
        pltpu.SemaphoreType.DMA,
        pltpu.SemaphoreType.DMA,
    ],
    compiler_params=pltpu.CompilerParams(use_tc_tiling_on_sc=False),
)(_embed_body)


def kernel(x, table):
    table_p = _prep(table.T)
    rows = _embed(x.T, table_p)
    r3 = rows.reshape(SEQ_J, SEQ_I, D_PAD)
    r3 = lax.slice(r3, (0, 0, 0), (SEQ_J, SEQ_I, D_MODEL))
    return r3.transpose(1, 0, 2)


# speedup vs baseline: 1.6760x; 1.6760x over previous
import functools

import jax
import jax.numpy as jnp
from jax import lax
from jax.experimental import pallas as pl
from jax.experimental.pallas import tpu as pltpu
from jax.experimental.pallas import tpu_sc as plsc

D_MODEL = 64
D_PAD = 128
N_ROWS = 1000000
SEQ_I = 4096
SEQ_J = 200
B = SEQ_I * SEQ_J
SCALE = 8.0
LANES = 16

NUM_CORES = 2
NUM_SUBCORES = 16
NW = NUM_CORES * NUM_SUBCORES

PREP_BLK = 4096
PREP_GRID = (N_ROWS + PREP_BLK - 1) // PREP_BLK


def _prep_body(t_ref, o_ref):
    rows = t_ref[...].T * SCALE
    o_ref[...] = jnp.concatenate(
        [rows, jnp.zeros((PREP_BLK, D_PAD - D_MODEL), jnp.float32)], axis=1)


_prep = pl.pallas_call(
    _prep_body,
    grid=(PREP_GRID,),
    in_specs=[pl.BlockSpec((D_MODEL, PREP_BLK), lambda r: (0, r))],
    out_specs=pl.BlockSpec((PREP_BLK, D_PAD), lambda r: (r, 0)),
    out_shape=jax.ShapeDtypeStruct((N_ROWS, D_PAD), jnp.float32),
)


CHUNK = 256
SUB = 128
NSUB = CHUNK // SUB
CHUNKS_PER_J = SEQ_I // CHUNK
N_UNITS = SEQ_J * CHUNKS_PER_J
N_CHUNKS = N_UNITS // NW


def _embed_body(x_hbm, table_hbm, out_hbm,
                idx0, idx1, rows0, rows1,
                isem0, isem1, gsem0, gsem1, ssem0, ssem1):
    wid = lax.axis_index("s") * NUM_CORES + lax.axis_index("c")
    u_base = wid * N_CHUNKS
    idx_v = (idx0, idx1)
    rows_v = (rows0, rows1)
    isem = (isem0, isem1)
    gsem = (gsem0, gsem1)
    ssem = (ssem0, ssem1)

    def unit_j_i0(c):
        g = u_base + c
        return g // CHUNKS_PER_J, (g % CHUNKS_PER_J) * CHUNK

    def fire_gathers(p):
        return [
            pltpu.async_copy(
                table_hbm.at[idx_v[p].at[pl.ds(s * SUB, SUB)]],
                rows_v[p].at[pl.ds(s * SUB, SUB)],
                gsem[p],
            )
            for s in range(NSUB)
        ]

    def wait_gathers(p):
        for s in range(NSUB):
            pltpu.make_async_copy(
                table_hbm.at[idx_v[p].at[pl.ds(s * SUB, SUB)]],
                rows_v[p].at[pl.ds(s * SUB, SUB)],
                gsem[p],
            ).wait()

    def fire_idx(c, p):
        j, i0 = unit_j_i0(c)
        pltpu.async_copy(x_hbm.at[j, pl.ds(i0, CHUNK)], idx_v[p], isem[p])

    def wait_idx(p):
        pltpu.make_async_copy(x_hbm.at[0, pl.ds(0, CHUNK)], idx_v[p],
                              isem[p]).wait()

    def fire_scatter(c, p):
        j, i0 = unit_j_i0(c)
        pltpu.async_copy(rows_v[p],
                         out_hbm.at[pl.ds(j * SEQ_I + i0, CHUNK)],
                         ssem[p])

    def wait_scatter(p):
        pltpu.make_async_copy(rows_v[p],
                              out_hbm.at[pl.ds(0, CHUNK)],
                              ssem[p]).wait()

    def step(c, p, wait_prev_scatter=True, prefetch_gather=True,
             prefetch_idx=True):
        q = 1 - p
        wait_gathers(p)
        if prefetch_gather:
            if wait_prev_scatter:
                wait_scatter(q)
            wait_idx(q)
            fire_gathers(q)
            if prefetch_idx:
                fire_idx(c + 2, p)
        fire_scatter(c, p)

    j0, i00 = unit_j_i0(0)
    pltpu.sync_copy(x_hbm.at[j0, pl.ds(i00, CHUNK)], idx_v[0])
    fire_gathers(0)
    fire_idx(1, 1)

    step(0, 0, wait_prev_scatter=False)
    step(1, 1)

    def pair_body(k, carry):
        c = 2 * k
        step(c, 0)
        step(c + 1, 1)
        return carry

    lax.fori_loop(1, N_CHUNKS // 2 - 1, pair_body, 0)

    step(N_CHUNKS - 2, 0, prefetch_idx=False)
    step(N_CHUNKS - 1, 1, prefetch_gather=False)

    wait_scatter(0)
    wait_scatter(1)


_embed = functools.partial(
    pl.kernel,
    out_type=jax.ShapeDtypeStruct((B, D_PAD), jnp.float32),
    mesh=plsc.VectorSubcoreMesh(
        core_axis_name="c",
        subcore_axis_name="s",
        num_cores=NUM_CORES,
        num_subcores=NUM_SUBCORES,
    ),
    scratch_types=[
        pltpu.VMEM((CHUNK,), jnp.int32),
        pltpu.VMEM((CHUNK,), jnp.int32),
        pltpu.VMEM((CHUNK, D_PAD), jnp.float32),
        pltpu.VMEM((CHUNK, D_PAD), jnp.float32),
        pltpu.SemaphoreType.DMA,
        pltpu.SemaphoreType.DMA,
        pltpu.SemaphoreType.DMA,
        pltpu.SemaphoreType.DMA,
        pltpu.SemaphoreType.DMA,
        pltpu.SemaphoreType.DMA,
    ],
    compiler_params=pltpu.CompilerParams(use_tc_tiling_on_sc=False),
)(_embed_body)


def kernel(x, table):
    table_p = _prep(table.T)
    rows = _embed(x.T, table_p)
    r3 = rows.reshape(SEQ_J, SEQ_I, D_PAD)
    r3 = lax.slice(r3, (0, 0, 0), (SEQ_J, SEQ_I, D_MODEL))
    return r3.transpose(1, 0, 2)


# speedup vs baseline: 1.8445x; 1.1005x over previous
import functools

import jax
import jax.numpy as jnp
from jax import lax
from jax.experimental import pallas as pl
from jax.experimental.pallas import tpu as pltpu
from jax.experimental.pallas import tpu_sc as plsc

D_MODEL = 64
D_PAD = 128
N_ROWS = 1000000
SEQ_I = 4096
SEQ_J = 200
B = SEQ_I * SEQ_J
SCALE = 8.0
LANES = 16

NUM_CORES = 2
NUM_SUBCORES = 16
NW = NUM_CORES * NUM_SUBCORES

PREP_BLK = 8192
PREP_GRID = (N_ROWS + PREP_BLK - 1) // PREP_BLK


def _prep_body(t_ref, o_ref):
    rows = t_ref[...].T * SCALE
    o_ref[...] = jnp.concatenate(
        [rows, jnp.zeros((PREP_BLK, D_PAD - D_MODEL), jnp.float32)], axis=1)


_prep = pl.pallas_call(
    _prep_body,
    grid=(PREP_GRID,),
    in_specs=[pl.BlockSpec((D_MODEL, PREP_BLK), lambda r: (0, r))],
    out_specs=pl.BlockSpec((PREP_BLK, D_PAD), lambda r: (r, 0)),
    out_shape=jax.ShapeDtypeStruct((N_ROWS, D_PAD), jnp.float32),
)


CHUNK = 256
SUB = 128
NSUB = CHUNK // SUB
CHUNKS_PER_J = SEQ_I // CHUNK
N_UNITS = SEQ_J * CHUNKS_PER_J
N_CHUNKS = N_UNITS // NW


def _embed_body(x_hbm, table_hbm, out_hbm,
                idx0, idx1, rows0, rows1,
                isem0, isem1, gsem0, gsem1, ssem0, ssem1):
    wid = lax.axis_index("s") * NUM_CORES + lax.axis_index("c")
    u_base = wid * N_CHUNKS
    idx_v = (idx0, idx1)
    rows_v = (rows0, rows1)
    isem = (isem0, isem1)
    gsem = (gsem0, gsem1)
    ssem = (ssem0, ssem1)

    def unit_j_i0(c):
        g = u_base + c
        return g // CHUNKS_PER_J, (g % CHUNKS_PER_J) * CHUNK

    def fire_gathers(p):
        return [
            pltpu.async_copy(
                table_hbm.at[idx_v[p].at[pl.ds(s * SUB, SUB)]],
                rows_v[p].at[pl.ds(s * SUB, SUB)],
                gsem[p],
            )
            for s in range(NSUB)
        ]

    def wait_gathers(p):
        for s in range(NSUB):
            pltpu.make_async_copy(
                table_hbm.at[idx_v[p].at[pl.ds(s * SUB, SUB)]],
                rows_v[p].at[pl.ds(s * SUB, SUB)],
                gsem[p],
            ).wait()

    def fire_idx(c, p):
        j, i0 = unit_j_i0(c)
        pltpu.async_copy(x_hbm.at[j, pl.ds(i0, CHUNK)], idx_v[p], isem[p])

    def wait_idx(p):
        pltpu.make_async_copy(x_hbm.at[0, pl.ds(0, CHUNK)], idx_v[p],
                              isem[p]).wait()

    def fire_scatter(c, p):
        j, i0 = unit_j_i0(c)
        pltpu.async_copy(rows_v[p],
                         out_hbm.at[pl.ds(j * SEQ_I + i0, CHUNK)],
                         ssem[p])

    def wait_scatter(p):
        pltpu.make_async_copy(rows_v[p],
                              out_hbm.at[pl.ds(0, CHUNK)],
                              ssem[p]).wait()

    def step(c, p, wait_prev_scatter=True, prefetch_gather=True,
             prefetch_idx=True):
        q = 1 - p
        wait_gathers(p)
        if prefetch_gather:
            if wait_prev_scatter:
                wait_scatter(q)
            wait_idx(q)
            fire_gathers(q)
            if prefetch_idx:
                fire_idx(c + 2, p)
        fire_scatter(c, p)

    j0, i00 = unit_j_i0(0)
    pltpu.sync_copy(x_hbm.at[j0, pl.ds(i00, CHUNK)], idx_v[0])
    fire_gathers(0)
    fire_idx(1, 1)

    step(0, 0, wait_prev_scatter=False)
    step(1, 1)

    def pair_body(k, carry):
        c = 2 * k
        step(c, 0)
        step(c + 1, 1)
        return carry

    lax.fori_loop(1, N_CHUNKS // 2 - 1, pair_body, 0)

    step(N_CHUNKS - 2, 0, prefetch_idx=False)
    step(N_CHUNKS - 1, 1, prefetch_gather=False)

    wait_scatter(0)
    wait_scatter(1)


_embed = functools.partial(
    pl.kernel,
    out_type=jax.ShapeDtypeStruct((B, D_PAD), jnp.float32),
    mesh=plsc.VectorSubcoreMesh(
        core_axis_name="c",
        subcore_axis_name="s",
        num_cores=NUM_CORES,
        num_subcores=NUM_SUBCORES,
    ),
    scratch_types=[
        pltpu.VMEM((CHUNK,), jnp.int32),
        pltpu.VMEM((CHUNK,), jnp.int32),
        pltpu.VMEM((CHUNK, D_PAD), jnp.float32),
        pltpu.VMEM((CHUNK, D_PAD), jnp.float32),
        pltpu.SemaphoreType.DMA,
        pltpu.SemaphoreType.DMA,
        pltpu.SemaphoreType.DMA,
        pltpu.SemaphoreType.DMA,
        pltpu.SemaphoreType.DMA,
        pltpu.SemaphoreType.DMA,
    ],
    compiler_params=pltpu.CompilerParams(use_tc_tiling_on_sc=False),
)(_embed_body)


def kernel(x, table):
    table_p = _prep(table.T)
    rows = _embed(x.T, table_p)
    r3 = rows.reshape(SEQ_J, SEQ_I, D_PAD)
    r3 = lax.slice(r3, (0, 0, 0), (SEQ_J, SEQ_I, D_MODEL))
    return r3.transpose(1, 0, 2)


# speedup vs baseline: 1.8957x; 1.0277x over previous
import functools

import jax
import jax.numpy as jnp
from jax import lax
from jax.experimental import pallas as pl
from jax.experimental.pallas import tpu as pltpu
from jax.experimental.pallas import tpu_sc as plsc

D_MODEL = 64
D_PAD = 128
N_ROWS = 1000000
SEQ_I = 4096
SEQ_J = 200
B = SEQ_I * SEQ_J
SCALE = 8.0
LANES = 16

NUM_CORES = 2
NUM_SUBCORES = 16
NW = NUM_CORES * NUM_SUBCORES

PREP_BLK = 16384
PREP_GRID = (N_ROWS + PREP_BLK - 1) // PREP_BLK


def _prep_body(t_ref, o_ref):
    rows = t_ref[...].T * SCALE
    o_ref[...] = jnp.concatenate(
        [rows, jnp.zeros((PREP_BLK, D_PAD - D_MODEL), jnp.float32)], axis=1)


_prep = pl.pallas_call(
    _prep_body,
    grid=(PREP_GRID,),
    in_specs=[pl.BlockSpec((D_MODEL, PREP_BLK), lambda r: (0, r))],
    out_specs=pl.BlockSpec((PREP_BLK, D_PAD), lambda r: (r, 0)),
    out_shape=jax.ShapeDtypeStruct((N_ROWS, D_PAD), jnp.float32),
)


CHUNK = 256
SUB = 128
NSUB = CHUNK // SUB
CHUNKS_PER_J = SEQ_I // CHUNK
N_UNITS = SEQ_J * CHUNKS_PER_J
N_CHUNKS = N_UNITS // NW


def _embed_body(x_hbm, table_hbm, out_hbm,
                idx0, idx1, rows0, rows1,
                isem0, isem1, gsem0, gsem1, ssem0, ssem1):
    wid = lax.axis_index("s") * NUM_CORES + lax.axis_index("c")
    u_base = wid * N_CHUNKS
    idx_v = (idx0, idx1)
    rows_v = (rows0, rows1)
    isem = (isem0, isem1)
    gsem = (gsem0, gsem1)
    ssem = (ssem0, ssem1)

    def unit_j_i0(c):
        g = u_base + c
        return g // CHUNKS_PER_J, (g % CHUNKS_PER_J) * CHUNK

    def fire_gathers(p):
        return [
            pltpu.async_copy(
                table_hbm.at[idx_v[p].at[pl.ds(s * SUB, SUB)]],
                rows_v[p].at[pl.ds(s * SUB, SUB)],
                gsem[p],
            )
            for s in range(NSUB)
        ]

    def wait_gathers(p):
        for s in range(NSUB):
            pltpu.make_async_copy(
                table_hbm.at[idx_v[p].at[pl.ds(s * SUB, SUB)]],
                rows_v[p].at[pl.ds(s * SUB, SUB)],
                gsem[p],
            ).wait()

    def fire_idx(c, p):
        j, i0 = unit_j_i0(c)
        pltpu.async_copy(x_hbm.at[j, pl.ds(i0, CHUNK)], idx_v[p], isem[p])

    def wait_idx(p):
        pltpu.make_async_copy(x_hbm.at[0, pl.ds(0, CHUNK)], idx_v[p],
                              isem[p]).wait()

    def fire_scatter(c, p):
        j, i0 = unit_j_i0(c)
        pltpu.async_copy(rows_v[p],
                         out_hbm.at[pl.ds(j * SEQ_I + i0, CHUNK)],
                         ssem[p])

    def wait_scatter(p):
        pltpu.make_async_copy(rows_v[p],
                              out_hbm.at[pl.ds(0, CHUNK)],
                              ssem[p]).wait()

    def step(c, p, wait_prev_scatter=True, prefetch_gather=True,
             prefetch_idx=True):
        q = 1 - p
        wait_gathers(p)
        if prefetch_gather:
            if wait_prev_scatter:
                wait_scatter(q)
            wait_idx(q)
            fire_gathers(q)
            if prefetch_idx:
                fire_idx(c + 2, p)
        fire_scatter(c, p)

    j0, i00 = unit_j_i0(0)
    pltpu.sync_copy(x_hbm.at[j0, pl.ds(i00, CHUNK)], idx_v[0])
    fire_gathers(0)
    fire_idx(1, 1)

    step(0, 0, wait_prev_scatter=False)
    step(1, 1)

    def pair_body(k, carry):
        c = 2 * k
        step(c, 0)
        step(c + 1, 1)
        return carry

    lax.fori_loop(1, N_CHUNKS // 2 - 1, pair_body, 0)

    step(N_CHUNKS - 2, 0, prefetch_idx=False)
    step(N_CHUNKS - 1, 1, prefetch_gather=False)

    wait_scatter(0)
    wait_scatter(1)


_embed = functools.partial(
    pl.kernel,
    out_type=jax.ShapeDtypeStruct((B, D_PAD), jnp.float32),
    mesh=plsc.VectorSubcoreMesh(
        core_axis_name="c",
        subcore_axis_name="s",
        num_cores=NUM_CORES,
        num_subcores=NUM_SUBCORES,
    ),
    scratch_types=[
        pltpu.VMEM((CHUNK,), jnp.int32),
        pltpu.VMEM((CHUNK,), jnp.int32),
        pltpu.VMEM((CHUNK, D_PAD), jnp.float32),
        pltpu.VMEM((CHUNK, D_PAD), jnp.float32),
        pltpu.SemaphoreType.DMA,
        pltpu.SemaphoreType.DMA,
        pltpu.SemaphoreType.DMA,
        pltpu.SemaphoreType.DMA,
        pltpu.SemaphoreType.DMA,
        pltpu.SemaphoreType.DMA,
    ],
    compiler_params=pltpu.CompilerParams(use_tc_tiling_on_sc=False),
)(_embed_body)


def kernel(x, table):
    table_p = _prep(table.T)
    rows = _embed(x.T, table_p)
    r3 = rows.reshape(SEQ_J, SEQ_I, D_PAD)
    r3 = lax.slice(r3, (0, 0, 0), (SEQ_J, SEQ_I, D_MODEL))
    return r3.transpose(1, 0, 2)


# speedup vs baseline: 1.9169x; 1.0112x over previous
import functools

import jax
import jax.numpy as jnp
from jax import lax
from jax.experimental import pallas as pl
from jax.experimental.pallas import tpu as pltpu
from jax.experimental.pallas import tpu_sc as plsc

D_MODEL = 64
D_PAD = 128
N_ROWS = 1000000
SEQ_I = 4096
SEQ_J = 200
B = SEQ_I * SEQ_J
SCALE = 8.0
LANES = 16

NUM_CORES = 2
NUM_SUBCORES = 16
NW = NUM_CORES * NUM_SUBCORES

PREP_BLK = 32768
PREP_GRID = (N_ROWS + PREP_BLK - 1) // PREP_BLK


def _prep_body(t_ref, o_ref):
    rows = t_ref[...].T * SCALE
    o_ref[...] = jnp.concatenate(
        [rows, jnp.zeros((PREP_BLK, D_PAD - D_MODEL), jnp.float32)], axis=1)


_prep = pl.pallas_call(
    _prep_body,
    grid=(PREP_GRID,),
    in_specs=[pl.BlockSpec((D_MODEL, PREP_BLK), lambda r: (0, r))],
    out_specs=pl.BlockSpec((PREP_BLK, D_PAD), lambda r: (r, 0)),
    out_shape=jax.ShapeDtypeStruct((N_ROWS, D_PAD), jnp.float32),
)


CHUNK = 256
SUB = 128
NSUB = CHUNK // SUB
CHUNKS_PER_J = SEQ_I // CHUNK
N_UNITS = SEQ_J * CHUNKS_PER_J
N_CHUNKS = N_UNITS // NW


def _embed_body(x_hbm, table_hbm, out_hbm,
                idx0, idx1, rows0, rows1,
                isem0, isem1, gsem0, gsem1, ssem0, ssem1):
    wid = lax.axis_index("s") * NUM_CORES + lax.axis_index("c")
    u_base = wid * N_CHUNKS
    idx_v = (idx0, idx1)
    rows_v = (rows0, rows1)
    isem = (isem0, isem1)
    gsem = (gsem0, gsem1)
    ssem = (ssem0, ssem1)

    def unit_j_i0(c):
        g = u_base + c
        return g // CHUNKS_PER_J, (g % CHUNKS_PER_J) * CHUNK

    def fire_gathers(p):
        return [
            pltpu.async_copy(
                table_hbm.at[idx_v[p].at[pl.ds(s * SUB, SUB)]],
                rows_v[p].at[pl.ds(s * SUB, SUB)],
                gsem[p],
            )
            for s in range(NSUB)
        ]

    def wait_gathers(p):
        for s in range(NSUB):
            pltpu.make_async_copy(
                table_hbm.at[idx_v[p].at[pl.ds(s * SUB, SUB)]],
                rows_v[p].at[pl.ds(s * SUB, SUB)],
                gsem[p],
            ).wait()

    def fire_idx(c, p):
        j, i0 = unit_j_i0(c)
        pltpu.async_copy(x_hbm.at[j, pl.ds(i0, CHUNK)], idx_v[p], isem[p])

    def wait_idx(p):
        pltpu.make_async_copy(x_hbm.at[0, pl.ds(0, CHUNK)], idx_v[p],
                              isem[p]).wait()

    def fire_scatter(c, p):
        j, i0 = unit_j_i0(c)
        pltpu.async_copy(rows_v[p],
                         out_hbm.at[pl.ds(j * SEQ_I + i0, CHUNK)],
                         ssem[p])

    def wait_scatter(p):
        pltpu.make_async_copy(rows_v[p],
                              out_hbm.at[pl.ds(0, CHUNK)],
                              ssem[p]).wait()

    def step(c, p, wait_prev_scatter=True, prefetch_gather=True,
             prefetch_idx=True):
        q = 1 - p
        wait_gathers(p)
        if prefetch_gather:
            if wait_prev_scatter:
                wait_scatter(q)
            wait_idx(q)
            fire_gathers(q)
            if prefetch_idx:
                fire_idx(c + 2, p)
        fire_scatter(c, p)

    j0, i00 = unit_j_i0(0)
    pltpu.sync_copy(x_hbm.at[j0, pl.ds(i00, CHUNK)], idx_v[0])
    fire_gathers(0)
    fire_idx(1, 1)

    step(0, 0, wait_prev_scatter=False)
    step(1, 1)

    def pair_body(k, carry):
        c = 2 * k
        step(c, 0)
        step(c + 1, 1)
        return carry

    lax.fori_loop(1, N_CHUNKS // 2 - 1, pair_body, 0)

    step(N_CHUNKS - 2, 0, prefetch_idx=False)
    step(N_CHUNKS - 1, 1, prefetch_gather=False)

    wait_scatter(0)
    wait_scatter(1)


_embed = functools.partial(
    pl.kernel,
    out_type=jax.ShapeDtypeStruct((B, D_PAD), jnp.float32),
    mesh=plsc.VectorSubcoreMesh(
        core_axis_name="c",
        subcore_axis_name="s",
        num_cores=NUM_CORES,
        num_subcores=NUM_SUBCORES,
    ),
    scratch_types=[
        pltpu.VMEM((CHUNK,), jnp.int32),
        pltpu.VMEM((CHUNK,), jnp.int32),
        pltpu.VMEM((CHUNK, D_PAD), jnp.float32),
        pltpu.VMEM((CHUNK, D_PAD), jnp.float32),
        pltpu.SemaphoreType.DMA,
        pltpu.SemaphoreType.DMA,
        pltpu.SemaphoreType.DMA,
        pltpu.SemaphoreType.DMA,
        pltpu.SemaphoreType.DMA,
        pltpu.SemaphoreType.DMA,
    ],
    compiler_params=pltpu.CompilerParams(use_tc_tiling_on_sc=False),
)(_embed_body)


def kernel(x, table):
    table_p = _prep(table.T)
    rows = _embed(x.T, table_p)
    r3 = rows.reshape(SEQ_J, SEQ_I, D_PAD)
    r3 = lax.slice(r3, (0, 0, 0), (SEQ_J, SEQ_I, D_MODEL))
    return r3.transpose(1, 0, 2)
